# baseline (speedup 1.0000x reference)
import functools

import jax
import jax.numpy as jnp
from jax import lax
from jax.experimental import pallas as pl
from jax.experimental.pallas import tpu as pltpu
from jax.experimental.pallas import tpu_sc as plsc

NUM_CLASSES = 1000
FEAT_DIM = 2048
SLOTS = 5
TOP_K = 3
B = 4096
NEG = -1e30

L = 16
NW = 32
QPW = B // NW
GP = 8
NG = QPW // GP
CH = FEAT_DIM // L
NU = 2
KR = SLOTS * GP
VR = TOP_K * GP
HQ = GP // 2
KH = SLOTS * HQ
VH = TOP_K * HQ


def _rsqrt_eps(x):
    xb = lax.bitcast_convert_type(x, jnp.int32)
    y = lax.bitcast_convert_type(
        jnp.int32(0x5F3759DF) - (xb >> 1), jnp.float32)
    for _ in range(3):
        y = y * (1.5 - 0.5 * x * y * y)
    return jnp.where(x < 1e-16, 1e8, y)


def _sc_body(keys_hbm, vals_hbm, labels_hbm, query_hbm, scores_hbm,
             ret_hbm, w_hbm,
             labels_v, scores_v, q_v, kbig, kidxA, kidxB, vidxA, vidxB,
             accm_d, accm_kk, accm_qq, sidx_v, attn_v, wout_v,
             semA, semB, semO):
    wid = lax.axis_index("s") * 2 + lax.axis_index("c")
    base = wid * QPW

    pltpu.sync_copy(labels_hbm.at[pl.ds(base, QPW)], labels_v)
    pltpu.sync_copy(scores_hbm, scores_v)

    lanes = lax.iota(jnp.int32, L)
    lane_q = lanes % GP
    mask16 = lanes < L
    mask8 = lanes < GP
    mask4 = lanes < HQ

    def group_body(g, _):
        lbase = g * GP
        gbase = base + lbase
        lv = plsc.load_gather(labels_v, [lbase + lane_q])

        lvA = plsc.load_gather(labels_v, [lbase + (lanes % HQ)])
        lvB = plsc.load_gather(labels_v, [lbase + HQ + (lanes % HQ)])
        for h, (idxr, lvh) in enumerate(((kidxA, lvA), (kidxB, lvB))):
            for b in range(2):
                ent = 16 * b + lanes
                srow = ent // HQ
                plsc.store_scatter(idxr, [ent], lvh + srow * NUM_CLASSES,
                                   mask=(mask16 if b == 0 else mask4))
        cpA = pltpu.async_copy(keys_hbm.at[kidxA],
                               kbig.at[pl.ds(0, KH)], semA)
        cpB = pltpu.async_copy(keys_hbm.at[kidxB],
                               kbig.at[pl.ds(KH, KH)], semB)
        @pl.when(g > 0)
        def _():
            pltpu.make_async_copy(q_v, ret_hbm.at[pl.ds(gbase - GP, GP)],
                                  semO).wait()
        pltpu.sync_copy(query_hbm.at[pl.ds(gbase, GP)], q_v)
        cpA.wait()

        def qi_body(qi, _):
            zero = jnp.zeros((L,), jnp.float32)
            nacc = (2 * SLOTS + 1) * NU

            def chunk(c, accs):
                accs = list(accs)
                for u in range(NU):
                    off = (c * NU + u) * L
                    qc = q_v[qi, pl.ds(off, L)]
                    for s in range(SLOTS):
                        kc = kbig[(qi // HQ) * KH + s * HQ + (qi % HQ),
                                  pl.ds(off, L)]
                        accs[s * NU + u] = accs[s * NU + u] + qc * kc
                        accs[(SLOTS + s) * NU + u] = (
                            accs[(SLOTS + s) * NU + u] + kc * kc)
                    accs[2 * SLOTS * NU + u] = (
                        accs[2 * SLOTS * NU + u] + qc * qc)
                return tuple(accs)

            accs = lax.fori_loop(0, CH // NU, chunk, (zero,) * nacc,
                                 unroll=2)
            for s in range(SLOTS):
                accm_d[pl.ds((s * GP + qi) * L, L)] = (
                    accs[s * NU] + accs[s * NU + 1])
                accm_kk[pl.ds((s * GP + qi) * L, L)] = (
                    accs[(SLOTS + s) * NU] + accs[(SLOTS + s) * NU + 1])
            accm_qq[pl.ds(qi * L, L)] = (
                accs[2 * SLOTS * NU] + accs[2 * SLOTS * NU + 1])
            return 0

        lax.fori_loop(0, HQ, qi_body, 0)
        cpB.wait()
        lax.fori_loop(HQ, GP, qi_body, 0)

        def colsum(m_ref, rowbase):
            accs = [jnp.zeros((L,), jnp.float32) for _ in range(4)]
            for c in range(L):
                accs[c % 4] = accs[c % 4] + plsc.load_gather(
                    m_ref, [(rowbase + lane_q) * L + c])
            return (accs[0] + accs[1]) + (accs[2] + accs[3])

        inv_qn = _rsqrt_eps(colsum(accm_qq, 0))
        comb = []
        rawsum = jnp.zeros((L,), jnp.float32)
        for s in range(SLOTS):
            sc = plsc.load_gather(scores_v, [lv * GP + s])
            rawsum = rawsum + sc
            d = colsum(accm_d, s * GP)
            kk = colsum(accm_kk, s * GP)
            comb.append(d * sc * _rsqrt_eps(kk) * inv_qn)

        hit = rawsum > 0.0

        work = list(comb)
        tops = []
        for k in range(TOP_K):
            m = work[0]
            for s in range(1, SLOTS):
                m = jnp.maximum(m, work[s])
            taken = jnp.zeros((L,), jnp.bool_)
            chosen = jnp.zeros((L,), jnp.int32)
            for s in range(SLOTS):
                isf = (work[s] == m) & (~taken)
                taken = taken | isf
                chosen = jnp.where(isf, jnp.int32(s), chosen)
                work[s] = jnp.where(isf, NEG, work[s])
            tops.append(m)
            sidx_v[pl.ds(k * L, L)] = chosen

        for h, (idxr, lvh, sm) in enumerate(
                ((vidxA, lvA, semA), (vidxB, lvB, semB))):
            ent = lanes
            kq = ent // HQ
            srow = plsc.load_gather(
                sidx_v, [kq * L + h * HQ + (lanes % HQ)])
            plsc.store_scatter(idxr, [ent], lvh + srow * NUM_CLASSES,
                               mask=lanes < VH)
        cpA2 = pltpu.async_copy(vals_hbm.at[vidxA],
                                kbig.at[pl.ds(0, VH)], semA)
        cpB2 = pltpu.async_copy(vals_hbm.at[vidxB],
                                kbig.at[pl.ds(VH, VH)], semB)

        exps = [jnp.exp((t - tops[0]) / 0.1) for t in tops]
        den = exps[0] + exps[1] + exps[2]
        for k in range(TOP_K):
            attn_v[pl.ds(k * L, L)] = jnp.where(hit, exps[k] / den, 0.0)

        w16 = jnp.where(hit, (tops[0] + tops[1] + tops[2]) / 3.0, 0.0)
        plsc.store_scatter(wout_v, [lbase + lanes], w16, mask=mask8)

        def qi_body2(qi, _):
            a = [plsc.load_gather(attn_v, [jnp.full((L,), k * L, jnp.int32)
                                           + qi])
                 for k in range(TOP_K)]

            def chunk(c, _):
                off = c * L
                hb = (qi // HQ) * VH
                qm = qi % HQ
                x = kbig[hb + qm, pl.ds(off, L)] * a[0]
                x = x + kbig[hb + HQ + qm, pl.ds(off, L)] * a[1]
                x = x + kbig[hb + 2 * HQ + qm, pl.ds(off, L)] * a[2]
                q_v[qi, pl.ds(off, L)] = x
                return 0

            lax.fori_loop(0, CH, chunk, 0, unroll=4)
            return 0

        cpA2.wait()
        lax.fori_loop(0, HQ, qi_body2, 0)
        cpB2.wait()
        lax.fori_loop(HQ, GP, qi_body2, 0)

        pltpu.async_copy(q_v, ret_hbm.at[pl.ds(gbase, GP)], semO)
        return 0

    lax.fori_loop(0, NG, group_body, 0)
    pltpu.make_async_copy(
        q_v, ret_hbm.at[pl.ds(base + QPW - GP, GP)], semO).wait()
    pltpu.sync_copy(wout_v, w_hbm.at[pl.ds(base, QPW)])


def kernel(query, labels, mem_keys, mem_vals, mem_scores):
    labels = labels.astype(jnp.int32)
    keys2d = mem_keys.transpose(1, 0, 2).reshape(SLOTS * NUM_CLASSES,
                                                 FEAT_DIM)
    vals2d = mem_vals.transpose(1, 0, 2).reshape(SLOTS * NUM_CLASSES,
                                                 FEAT_DIM)
    scores_pad = jnp.zeros((NUM_CLASSES, GP), jnp.float32)
    scores_pad = scores_pad.at[:, :SLOTS].set(mem_scores).reshape(-1)

    run = functools.partial(
        pl.kernel,
        out_type=[
            jax.ShapeDtypeStruct((B, FEAT_DIM), jnp.float32),
            jax.ShapeDtypeStruct((B,), jnp.float32),
        ],
        mesh=plsc.VectorSubcoreMesh(core_axis_name="c", subcore_axis_name="s"),
        compiler_params=pltpu.CompilerParams(needs_layout_passes=False, use_tc_tiling_on_sc=False),
        scratch_types=[
            pltpu.VMEM((QPW,), jnp.int32),
            pltpu.VMEM((NUM_CLASSES * GP,), jnp.float32),
            pltpu.VMEM((GP, FEAT_DIM), jnp.float32),
            pltpu.VMEM((KR, FEAT_DIM), jnp.float32),
            pltpu.VMEM((KH,), jnp.int32),
            pltpu.VMEM((KH,), jnp.int32),
            pltpu.VMEM((VH,), jnp.int32),
            pltpu.VMEM((VH,), jnp.int32),
            pltpu.VMEM((SLOTS * GP * L,), jnp.float32),
            pltpu.VMEM((SLOTS * GP * L,), jnp.float32),
            pltpu.VMEM((GP * L,), jnp.float32),
            pltpu.VMEM((TOP_K * L,), jnp.int32),
            pltpu.VMEM((TOP_K * L,), jnp.float32),
            pltpu.VMEM((QPW,), jnp.float32),
            pltpu.SemaphoreType.DMA,
            pltpu.SemaphoreType.DMA,
            pltpu.SemaphoreType.DMA,
        ],
    )(_sc_body)
    retrieved, weights = run(keys2d, vals2d, labels, query, scores_pad)
    return retrieved, weights


# speedup vs baseline: 1.1795x; 1.1795x over previous
import functools

import jax
import jax.numpy as jnp
from jax import lax
from jax.experimental import pallas as pl
from jax.experimental.pallas import tpu as pltpu
from jax.experimental.pallas import tpu_sc as plsc

NUM_CLASSES = 1000
FEAT_DIM = 2048
SLOTS = 5
TOP_K = 3
B = 4096
NEG = -1e30

L = 16
NW = 32
QPW = B // NW
GP = 8
NG = QPW // GP
CH = FEAT_DIM // L
NU = 2
KR = SLOTS * GP
VR = TOP_K * GP
HQ = GP // 2
KH = 24
VH = 16


def _rsqrt_eps(x):
    xb = lax.bitcast_convert_type(x, jnp.int32)
    y = lax.bitcast_convert_type(
        jnp.int32(0x5F3759DF) - (xb >> 1), jnp.float32)
    for _ in range(3):
        y = y * (1.5 - 0.5 * x * y * y)
    return jnp.where(x < 1e-16, 1e8, y)


def _sc_body(keys_hbm, vals_hbm, labels_hbm, query_hbm, scores_hbm,
             ret_hbm, w_hbm,
             labels_v, scores_v, q_v, kbig, kidxA, kidxB, vidxA, vidxB,
             accm_d, accm_kk, accm_qq, sidx_v, attn_v, wout_v,
             semA, semB, semO):
    wid = lax.axis_index("s") * 2 + lax.axis_index("c")
    base = wid * QPW

    pltpu.sync_copy(labels_hbm.at[pl.ds(base, QPW)], labels_v)
    pltpu.sync_copy(scores_hbm, scores_v)

    lanes = lax.iota(jnp.int32, L)
    lane_q = lanes % GP
    mask16 = lanes < L
    mask8 = lanes < GP
    mask4 = lanes < HQ

    def group_body(g, _):
        lbase = g * GP
        gbase = base + lbase
        lv = plsc.load_gather(labels_v, [lbase + lane_q])

        lvA = plsc.load_gather(labels_v, [lbase + (lanes % HQ)])
        lvB = plsc.load_gather(labels_v, [lbase + HQ + (lanes % HQ)])
        for h, (idxr, lvh) in enumerate(((kidxA, lvA), (kidxB, lvB))):
            for b in range(2):
                ent = 16 * b + lanes
                srow = jnp.minimum(ent // HQ, SLOTS - 1)
                plsc.store_scatter(idxr, [ent], lvh + srow * NUM_CLASSES,
                                   mask=(mask16 if b == 0 else mask8))
        cpA = pltpu.async_copy(keys_hbm.at[kidxA],
                               kbig.at[pl.ds(0, KH)], semA)
        cpB = pltpu.async_copy(keys_hbm.at[kidxB],
                               kbig.at[pl.ds(KH, KH)], semB)
        @pl.when(g > 0)
        def _():
            pltpu.make_async_copy(q_v, ret_hbm.at[pl.ds(gbase - GP, GP)],
                                  semO).wait()
        pltpu.sync_copy(query_hbm.at[pl.ds(gbase, GP)], q_v)
        cpA.wait()

        def qi_body(qi, _):
            zero = jnp.zeros((L,), jnp.float32)
            nacc = (2 * SLOTS + 1) * NU

            def chunk(c, accs):
                accs = list(accs)
                for u in range(NU):
                    off = (c * NU + u) * L
                    qc = q_v[qi, pl.ds(off, L)]
                    for s in range(SLOTS):
                        kc = kbig[(qi // HQ) * KH + s * HQ + (qi % HQ),
                                  pl.ds(off, L)]
                        accs[s * NU + u] = accs[s * NU + u] + qc * kc
                        accs[(SLOTS + s) * NU + u] = (
                            accs[(SLOTS + s) * NU + u] + kc * kc)
                    accs[2 * SLOTS * NU + u] = (
                        accs[2 * SLOTS * NU + u] + qc * qc)
                return tuple(accs)

            accs = lax.fori_loop(0, CH // NU, chunk, (zero,) * nacc,
                                 unroll=2)
            for s in range(SLOTS):
                accm_d[pl.ds((s * GP + qi) * L, L)] = (
                    accs[s * NU] + accs[s * NU + 1])
                accm_kk[pl.ds((s * GP + qi) * L, L)] = (
                    accs[(SLOTS + s) * NU] + accs[(SLOTS + s) * NU + 1])
            accm_qq[pl.ds(qi * L, L)] = (
                accs[2 * SLOTS * NU] + accs[2 * SLOTS * NU + 1])
            return 0

        lax.fori_loop(0, HQ, qi_body, 0)
        cpB.wait()
        lax.fori_loop(HQ, GP, qi_body, 0)

        def colsum(m_ref, rowbase):
            accs = [jnp.zeros((L,), jnp.float32) for _ in range(4)]
            for c in range(L):
                accs[c % 4] = accs[c % 4] + plsc.load_gather(
                    m_ref, [(rowbase + lane_q) * L + c])
            return (accs[0] + accs[1]) + (accs[2] + accs[3])

        inv_qn = _rsqrt_eps(colsum(accm_qq, 0))
        comb = []
        rawsum = jnp.zeros((L,), jnp.float32)
        for s in range(SLOTS):
            sc = plsc.load_gather(scores_v, [lv * GP + s])
            rawsum = rawsum + sc
            d = colsum(accm_d, s * GP)
            kk = colsum(accm_kk, s * GP)
            comb.append(d * sc * _rsqrt_eps(kk) * inv_qn)

        hit = rawsum > 0.0

        work = list(comb)
        tops = []
        for k in range(TOP_K):
            m = work[0]
            for s in range(1, SLOTS):
                m = jnp.maximum(m, work[s])
            taken = jnp.zeros((L,), jnp.bool_)
            chosen = jnp.zeros((L,), jnp.int32)
            for s in range(SLOTS):
                isf = (work[s] == m) & (~taken)
                taken = taken | isf
                chosen = jnp.where(isf, jnp.int32(s), chosen)
                work[s] = jnp.where(isf, NEG, work[s])
            tops.append(m)
            sidx_v[pl.ds(k * L, L)] = chosen

        for h, (idxr, lvh, sm) in enumerate(
                ((vidxA, lvA, semA), (vidxB, lvB, semB))):
            ent = lanes
            kq = jnp.minimum(ent // HQ, TOP_K - 1)
            srow = plsc.load_gather(
                sidx_v, [kq * L + h * HQ + (lanes % HQ)])
            plsc.store_scatter(idxr, [ent], lvh + srow * NUM_CLASSES,
                               mask=mask16)
        cpA2 = pltpu.async_copy(vals_hbm.at[vidxA],
                                kbig.at[pl.ds(0, VH)], semA)
        cpB2 = pltpu.async_copy(vals_hbm.at[vidxB],
                                kbig.at[pl.ds(VH, VH)], semB)

        exps = [jnp.exp((t - tops[0]) / 0.1) for t in tops]
        den = exps[0] + exps[1] + exps[2]
        for k in range(TOP_K):
            attn_v[pl.ds(k * L, L)] = jnp.where(hit, exps[k] / den, 0.0)

        w16 = jnp.where(hit, (tops[0] + tops[1] + tops[2]) / 3.0, 0.0)
        plsc.store_scatter(wout_v, [lbase + lanes], w16, mask=mask8)

        def qi_body2(qi, _):
            a = [plsc.load_gather(attn_v, [jnp.full((L,), k * L, jnp.int32)
                                           + qi])
                 for k in range(TOP_K)]

            def chunk(c, _):
                off = c * L
                hb = (qi // HQ) * VH
                qm = qi % HQ
                x = kbig[hb + qm, pl.ds(off, L)] * a[0]
                x = x + kbig[hb + HQ + qm, pl.ds(off, L)] * a[1]
                x = x + kbig[hb + 2 * HQ + qm, pl.ds(off, L)] * a[2]
                q_v[qi, pl.ds(off, L)] = x
                return 0

            lax.fori_loop(0, CH, chunk, 0, unroll=4)
            return 0

        cpA2.wait()
        lax.fori_loop(0, HQ, qi_body2, 0)
        cpB2.wait()
        lax.fori_loop(HQ, GP, qi_body2, 0)

        pltpu.async_copy(q_v, ret_hbm.at[pl.ds(gbase, GP)], semO)
        return 0

    lax.fori_loop(0, NG, group_body, 0)
    pltpu.make_async_copy(
        q_v, ret_hbm.at[pl.ds(base + QPW - GP, GP)], semO).wait()
    pltpu.sync_copy(wout_v, w_hbm.at[pl.ds(base, QPW)])


def kernel(query, labels, mem_keys, mem_vals, mem_scores):
    labels = labels.astype(jnp.int32)
    keys2d = mem_keys.transpose(1, 0, 2).reshape(SLOTS * NUM_CLASSES,
                                                 FEAT_DIM)
    vals2d = mem_vals.transpose(1, 0, 2).reshape(SLOTS * NUM_CLASSES,
                                                 FEAT_DIM)
    scores_pad = jnp.zeros((NUM_CLASSES, GP), jnp.float32)
    scores_pad = scores_pad.at[:, :SLOTS].set(mem_scores).reshape(-1)

    run = functools.partial(
        pl.kernel,
        out_type=[
            jax.ShapeDtypeStruct((B, FEAT_DIM), jnp.float32),
            jax.ShapeDtypeStruct((B,), jnp.float32),
        ],
        mesh=plsc.VectorSubcoreMesh(core_axis_name="c", subcore_axis_name="s"),
        compiler_params=pltpu.CompilerParams(needs_layout_passes=False),
        scratch_types=[
            pltpu.VMEM((QPW,), jnp.int32),
            pltpu.VMEM((NUM_CLASSES * GP,), jnp.float32),
            pltpu.VMEM((GP, FEAT_DIM), jnp.float32),
            pltpu.VMEM((2 * KH, FEAT_DIM), jnp.float32),
            pltpu.VMEM((KH,), jnp.int32),
            pltpu.VMEM((KH,), jnp.int32),
            pltpu.VMEM((VH,), jnp.int32),
            pltpu.VMEM((VH,), jnp.int32),
            pltpu.VMEM((SLOTS * GP * L,), jnp.float32),
            pltpu.VMEM((SLOTS * GP * L,), jnp.float32),
            pltpu.VMEM((GP * L,), jnp.float32),
            pltpu.VMEM((TOP_K * L,), jnp.int32),
            pltpu.VMEM((TOP_K * L,), jnp.float32),
            pltpu.VMEM((QPW,), jnp.float32),
            pltpu.SemaphoreType.DMA,
            pltpu.SemaphoreType.DMA,
            pltpu.SemaphoreType.DMA,
        ],
    )(_sc_body)
    retrieved, weights = run(keys2d, vals2d, labels, query, scores_pad)
    return retrieved, weights


# speedup vs baseline: 1.2617x; 1.0696x over previous
import functools

import jax
import jax.numpy as jnp
from jax import lax
from jax.experimental import pallas as pl
from jax.experimental.pallas import tpu as pltpu
from jax.experimental.pallas import tpu_sc as plsc

NUM_CLASSES = 1000
FEAT_DIM = 2048
SLOTS = 5
TOP_K = 3
B = 4096
NEG = -1e30

L = 16
NW = 32
QPW = B // NW
GP = 8
NG = QPW // GP
CH = FEAT_DIM // L
NU = 2
KR = SLOTS * GP
VR = TOP_K * GP


def _rsqrt_eps(x):
    xb = lax.bitcast_convert_type(x, jnp.int32)
    y = lax.bitcast_convert_type(
        jnp.int32(0x5F3759DF) - (xb >> 1), jnp.float32)
    for _ in range(3):
        y = y * (1.5 - 0.5 * x * y * y)
    return jnp.where(x < 1e-16, 1e8, y)


def _sc_body(keys_hbm, vals_hbm, labels_hbm, query_hbm, scores_hbm,
             ret_hbm, w_hbm,
             labels_v, scores_v, q_v, kbig, kidx, vidx,
             accm_d, accm_kk, accm_qq, sidx_v, attn_v, wout_v, sem, semO):
    wid = lax.axis_index("s") * 2 + lax.axis_index("c")
    base = wid * QPW

    pltpu.sync_copy(labels_hbm.at[pl.ds(base, QPW)], labels_v)
    pltpu.sync_copy(scores_hbm, scores_v)

    lanes = lax.iota(jnp.int32, L)
    lane_q = lanes % GP
    mask16 = lanes < L
    mask8 = lanes < GP

    def group_body(g, _):
        lbase = g * GP
        gbase = base + lbase
        lv = plsc.load_gather(labels_v, [lbase + lane_q])

        for b in range(3):
            ent = 16 * b + lanes
            srow = ent // GP
            plsc.store_scatter(kidx, [ent], lv + srow * NUM_CLASSES,
                               mask=(mask16 if b < 2 else mask8))
        cp = pltpu.async_copy(keys_hbm.at[kidx], kbig, sem)

        @pl.when(g > 0)
        def _():
            pltpu.make_async_copy(
                q_v, ret_hbm.at[pl.ds(gbase - GP, GP)], semO).wait()
        pltpu.sync_copy(query_hbm.at[pl.ds(gbase, GP)], q_v)
        cp.wait()

        def qi_body(qi, _):
            zero = jnp.zeros((L,), jnp.float32)
            nacc = (2 * SLOTS + 1) * NU

            def chunk(c, accs):
                accs = list(accs)
                for u in range(NU):
                    off = (c * NU + u) * L
                    qc = q_v[qi, pl.ds(off, L)]
                    for s in range(SLOTS):
                        kc = kbig[s * GP + qi, pl.ds(off, L)]
                        accs[s * NU + u] = accs[s * NU + u] + qc * kc
                        accs[(SLOTS + s) * NU + u] = (
                            accs[(SLOTS + s) * NU + u] + kc * kc)
                    accs[2 * SLOTS * NU + u] = (
                        accs[2 * SLOTS * NU + u] + qc * qc)
                return tuple(accs)

            accs = lax.fori_loop(0, CH // NU, chunk, (zero,) * nacc,
                                 unroll=2)
            for s in range(SLOTS):
                accm_d[pl.ds((s * GP + qi) * L, L)] = (
                    accs[s * NU] + accs[s * NU + 1])
                accm_kk[pl.ds((s * GP + qi) * L, L)] = (
                    accs[(SLOTS + s) * NU] + accs[(SLOTS + s) * NU + 1])
            accm_qq[pl.ds(qi * L, L)] = (
                accs[2 * SLOTS * NU] + accs[2 * SLOTS * NU + 1])
            return 0

        lax.fori_loop(0, GP, qi_body, 0)

        def colsum(m_ref, rowbase):
            accs = [jnp.zeros((L,), jnp.float32) for _ in range(4)]
            for c in range(L):
                accs[c % 4] = accs[c % 4] + plsc.load_gather(
                    m_ref, [(rowbase + lane_q) * L + c])
            return (accs[0] + accs[1]) + (accs[2] + accs[3])

        inv_qn = _rsqrt_eps(colsum(accm_qq, 0))
        comb = []
        rawsum = jnp.zeros((L,), jnp.float32)
        for s in range(SLOTS):
            sc = plsc.load_gather(scores_v, [lv * GP + s])
            rawsum = rawsum + sc
            d = colsum(accm_d, s * GP)
            kk = colsum(accm_kk, s * GP)
            comb.append(d * sc * _rsqrt_eps(kk) * inv_qn)

        hit = rawsum > 0.0

        work = list(comb)
        tops = []
        for k in range(TOP_K):
            m = work[0]
            for s in range(1, SLOTS):
                m = jnp.maximum(m, work[s])
            taken = jnp.zeros((L,), jnp.bool_)
            chosen = jnp.zeros((L,), jnp.int32)
            for s in range(SLOTS):
                isf = (work[s] == m) & (~taken)
                taken = taken | isf
                chosen = jnp.where(isf, jnp.int32(s), chosen)
                work[s] = jnp.where(isf, NEG, work[s])
            tops.append(m)
            sidx_v[pl.ds(k * L, L)] = chosen

        for b in range(2):
            ent = 16 * b + lanes
            kq = ent // GP
            srow = plsc.load_gather(sidx_v, [kq * L + lane_q])
            plsc.store_scatter(vidx, [ent], lv + srow * NUM_CLASSES,
                               mask=(mask16 if b == 0 else mask8))
        cpv = pltpu.async_copy(vals_hbm.at[vidx], kbig.at[pl.ds(0, VR)], sem)

        exps = [jnp.exp((t - tops[0]) / 0.1) for t in tops]
        den = exps[0] + exps[1] + exps[2]
        for k in range(TOP_K):
            attn_v[pl.ds(k * L, L)] = jnp.where(hit, exps[k] / den, 0.0)

        w16 = jnp.where(hit, (tops[0] + tops[1] + tops[2]) / 3.0, 0.0)
        plsc.store_scatter(wout_v, [lbase + lanes], w16, mask=mask8)

        cpv.wait()

        def qi_body2(qi, _):
            a = [plsc.load_gather(attn_v, [jnp.full((L,), k * L, jnp.int32)
                                           + qi])
                 for k in range(TOP_K)]

            def chunk(c, _):
                off = c * L
                x = kbig[qi, pl.ds(off, L)] * a[0]
                x = x + kbig[GP + qi, pl.ds(off, L)] * a[1]
                x = x + kbig[2 * GP + qi, pl.ds(off, L)] * a[2]
                q_v[qi, pl.ds(off, L)] = x
                return 0

            lax.fori_loop(0, CH, chunk, 0, unroll=4)
            return 0

        lax.fori_loop(0, GP, qi_body2, 0)

        pltpu.async_copy(q_v, ret_hbm.at[pl.ds(gbase, GP)], semO)
        return 0

    lax.fori_loop(0, NG, group_body, 0)
    pltpu.make_async_copy(
        q_v, ret_hbm.at[pl.ds(base + QPW - GP, GP)], semO).wait()
    pltpu.sync_copy(wout_v, w_hbm.at[pl.ds(base, QPW)])


def kernel(query, labels, mem_keys, mem_vals, mem_scores):
    labels = labels.astype(jnp.int32)
    keys2d = mem_keys.transpose(1, 0, 2).reshape(SLOTS * NUM_CLASSES,
                                                 FEAT_DIM)
    vals2d = mem_vals.transpose(1, 0, 2).reshape(SLOTS * NUM_CLASSES,
                                                 FEAT_DIM)
    scores_pad = jnp.zeros((NUM_CLASSES, GP), jnp.float32)
    scores_pad = scores_pad.at[:, :SLOTS].set(mem_scores).reshape(-1)

    run = functools.partial(
        pl.kernel,
        out_type=[
            jax.ShapeDtypeStruct((B, FEAT_DIM), jnp.float32),
            jax.ShapeDtypeStruct((B,), jnp.float32),
        ],
        mesh=plsc.VectorSubcoreMesh(core_axis_name="c", subcore_axis_name="s"),
        compiler_params=pltpu.CompilerParams(needs_layout_passes=False),
        scratch_types=[
            pltpu.VMEM((QPW,), jnp.int32),
            pltpu.VMEM((NUM_CLASSES * GP,), jnp.float32),
            pltpu.VMEM((GP, FEAT_DIM), jnp.float32),
            pltpu.VMEM((KR, FEAT_DIM), jnp.float32),
            pltpu.VMEM((KR,), jnp.int32),
            pltpu.VMEM((VR,), jnp.int32),
            pltpu.VMEM((SLOTS * GP * L,), jnp.float32),
            pltpu.VMEM((SLOTS * GP * L,), jnp.float32),
            pltpu.VMEM((GP * L,), jnp.float32),
            pltpu.VMEM((TOP_K * L,), jnp.int32),
            pltpu.VMEM((TOP_K * L,), jnp.float32),
            pltpu.VMEM((QPW,), jnp.float32),
            pltpu.SemaphoreType.DMA,
            pltpu.SemaphoreType.DMA,
        ],
    )(_sc_body)
    retrieved, weights = run(keys2d, vals2d, labels, query, scores_pad)
    return retrieved, weights
